# baseline (speedup 1.0000x reference)
import math
from functools import partial

import jax
import jax.numpy as jnp
from jax import lax
from jax.experimental import pallas as pl
from jax.experimental.pallas import tpu as pltpu
from jax.experimental.pallas import tpu_sc as plsc

B, S, E, H = 4, 4096, 1024, 16
DH = E // H
SEL = max(1, int(5 * math.log(S + 1)))
SEL = min(SEL, S)
SELP = 48
ROWS = H * SELP
SB = 1024
NSB = S // SB
T_STEPS = B * NSB + NSB
SCALE = 1.0 / math.sqrt(DH)
IDXW = 64
BF = jnp.bfloat16


def _head_mask(rows, cols, row_group, col_group, dtype):
    r = jax.lax.broadcasted_iota(jnp.int32, (rows, cols), 0) // row_group
    c = jax.lax.broadcasted_iota(jnp.int32, (rows, cols), 1) // col_group
    return (r == c).astype(dtype)


def _stats_kernel(x_ref, mean_ref, idx_ref, e_ref, cs_ref):
    b = pl.program_id(0)
    s = pl.program_id(1)
    xb = x_ref[0]
    colsum = jnp.sum(xb, axis=0).reshape(1, E)
    y = xb[:, 0:128] * xb[:, 0:128]
    for k in range(1, E // 128):
        xk = xb[:, k * 128:(k + 1) * 128]
        y = y + xk * xk
    energy = jnp.sum(y, axis=1).reshape(1, SB)

    @pl.when(s == 0)
    def _():
        cs_ref[...] = colsum

    @pl.when(s != 0)
    def _():
        cs_ref[...] += colsum

    e_ref[pl.ds(b, 1), pl.ds(s * SB, SB)] = energy

    @pl.when(s == NSB - 1)
    def _():
        mean_ref[0] = cs_ref[...] * (1.0 / S)

    @pl.when((b == B - 1) & (s == NSB - 1))
    def _():
        lane_s = jax.lax.broadcasted_iota(jnp.int32, (B, S), 1)
        lane_w = jax.lax.broadcasted_iota(jnp.int32, (B, IDXW), 1)

        def body(j, carry):
            e, acc = carry
            m = jnp.max(e, axis=1, keepdims=True)
            idxv = jnp.min(jnp.where(e == m, lane_s, S), axis=1, keepdims=True)
            acc = jnp.where(lane_w == j, idxv, acc)
            e = jnp.where(lane_s == idxv, -1.0, e)
            return e, acc

        _, acc = jax.lax.fori_loop(
            0, SEL, body, (e_ref[...], jnp.zeros((B, IDXW), jnp.int32)))
        idx_ref[...] = acc.reshape(B, 1, IDXW)


def _stats(x):
    return pl.pallas_call(
        _stats_kernel,
        grid=(B, NSB),
        in_specs=[pl.BlockSpec((1, SB, E), lambda b, s: (b, s, 0))],
        out_specs=[
            pl.BlockSpec((1, 1, E), lambda b, s: (b, 0, 0)),
            pl.BlockSpec((B, 1, IDXW), lambda b, s: (0, 0, 0)),
        ],
        out_shape=[
            jax.ShapeDtypeStruct((B, 1, E), jnp.float32),
            jax.ShapeDtypeStruct((B, 1, IDXW), jnp.int32),
        ],
        scratch_shapes=[
            pltpu.VMEM((B, S), jnp.float32),
            pltpu.VMEM((1, E), jnp.float32),
        ],
    )(x)


def _sc_gather(x, idx2d):
    mesh = plsc.VectorSubcoreMesh(core_axis_name="c", subcore_axis_name="s")

    @partial(
        pl.kernel, mesh=mesh,
        out_type=jax.ShapeDtypeStruct((B, IDXW, E), jnp.float32),
        scratch_types=[
            pltpu.VMEM((IDXW,), jnp.int32),
            pltpu.VMEM((IDXW, E), jnp.float32),
            pltpu.SemaphoreType.DMA,
        ],
    )
    def k(x_hbm, idx_hbm, out_hbm, idx_v, rows_v, sem):
        wid = lax.axis_index("s") * 2 + lax.axis_index("c")

        @pl.when(wid < B)
        def _():
            pltpu.sync_copy(idx_hbm.at[wid], idx_v)
            pltpu.async_copy(x_hbm.at[wid].at[idx_v], rows_v, sem).wait()
            pltpu.sync_copy(rows_v, out_hbm.at[wid])

    return k(x, idx2d)


def _bc(t):
    return jnp.minimum(t // NSB, B - 1)


def _bf(t):
    return jnp.maximum(t - NSB, 0) // NSB


def _sf(t):
    return jnp.maximum(t - NSB, 0) % NSB


def _mega_kernel(idxp_ref, xqin_ref, x_ref, mean_ref, idx_ref,
                 wq_ref, wk_ref, wv_ref, wo_ref, bq_ref, bv_ref, bo_ref,
                 fill_ref,
                 qt_ref, rows_ref, acc_ref, m_ref, l_ref,
                 wq16_ref, wk16_ref, wv16_ref, wo16_ref):
    t = pl.program_id(0)
    bc = _bc(t)
    s = t % NSB
    compute = t < B * NSB

    @pl.when(t == 0)
    def _():
        wq16_ref[...] = wq_ref[...].astype(BF)
        wk16_ref[...] = wk_ref[...].astype(BF)
        wv16_ref[...] = wv_ref[...].astype(BF)
        wo16_ref[...] = wo_ref[...].astype(BF)

    @pl.when(t >= NSB)
    def _():
        bf = _bf(t)
        sf = _sf(t)
        mean = mean_ref[0]
        delta = rows_ref[...] - mean
        row_g = jax.lax.broadcasted_iota(jnp.int32, (SB, SELP), 0) + sf * SB
        col_j = jax.lax.broadcasted_iota(jnp.int32, (SB, SELP), 1)
        idxv = idx_ref[bf, 0, :SELP].reshape(1, SELP)
        onehot = ((row_g == idxv) & (col_j < SEL)).astype(BF)
        fill_ref[0] = jnp.broadcast_to(mean, (SB, E)) + jax.lax.dot_general(
            onehot, delta.astype(BF), (((1,), (0,)), ((), ())),
            preferred_element_type=jnp.float32)

    @pl.when(compute & (s == 0))
    def _():
        q = jax.lax.dot_general(
            xqin_ref[0, :SELP, :].astype(BF), wq16_ref[...],
            (((1,), (1,)), ((), ())),
            preferred_element_type=jnp.float32) + bq_ref[...]
        qe = jnp.concatenate([q] * H, axis=0)
        qe = (qe * _head_mask(ROWS, E, SELP, DH, jnp.float32)).astype(BF)
        qt = jax.lax.dot_general(
            qe, wk16_ref[...], (((1,), (0,)), ((), ())),
            preferred_element_type=jnp.float32)
        qt_ref[...] = (qt * SCALE).astype(BF)

    @pl.when(compute)
    def _():
        xb16 = x_ref[0].astype(BF)
        scores = jax.lax.dot_general(
            qt_ref[...], xb16, (((1,), (1,)), ((), ())),
            preferred_element_type=jnp.float32)
        bmax = jnp.max(scores, axis=1, keepdims=True)

        @pl.when(s == 0)
        def _():
            p = jnp.exp(scores - bmax)
            m_ref[...] = bmax
            l_ref[...] = jnp.sum(p, axis=1, keepdims=True)
            acc_ref[...] = jax.lax.dot_general(
                p.astype(BF), xb16, (((1,), (0,)), ((), ())),
                preferred_element_type=jnp.float32)

        @pl.when(s != 0)
        def _():
            m_old = m_ref[...]
            m_new = jnp.maximum(m_old, bmax)
            alpha = jnp.exp(m_old - m_new)
            p = jnp.exp(scores - m_new)
            m_ref[...] = m_new
            l_ref[...] = l_ref[...] * alpha + jnp.sum(p, axis=1, keepdims=True)
            acc_ref[...] = acc_ref[...] * alpha + jax.lax.dot_general(
                p.astype(BF), xb16, (((1,), (0,)), ((), ())),
                preferred_element_type=jnp.float32)

        @pl.when(s == NSB - 1)
        def _():
            z = acc_ref[...] / l_ref[...]
            oh = jax.lax.dot_general(
                z.astype(BF), wv16_ref[...], (((1,), (1,)), ((), ())),
                preferred_element_type=jnp.float32)
            oh = oh * _head_mask(ROWS, E, SELP, DH, jnp.float32)
            folded = jnp.zeros((SELP, E), jnp.float32)
            for h in range(H):
                folded = folded + oh[h * SELP:(h + 1) * SELP, :]
            bvo = jax.lax.dot_general(
                bv_ref[...].astype(BF), wo16_ref[...], (((1,), (1,)), ((), ())),
                preferred_element_type=jnp.float32) + bo_ref[...]
            rows_ref[...] = jax.lax.dot_general(
                folded.astype(BF), wo16_ref[...], (((1,), (1,)), ((), ())),
                preferred_element_type=jnp.float32) + bvo


def _mega(xq, idx2d, mean, idx, W_qkv, W_out, bq, bv, bo, x):
    bspec = pl.BlockSpec((1, E), lambda t, p: (0, 0))
    return pl.pallas_call(
        _mega_kernel,
        grid_spec=pltpu.PrefetchScalarGridSpec(
            num_scalar_prefetch=1,
            grid=(T_STEPS,),
            in_specs=[
                pl.BlockSpec((1, IDXW, E), lambda t, p: (_bc(t), 0, 0)),
                pl.BlockSpec((1, SB, E),
                             lambda t, p: (_bc(t),
                                           jnp.minimum(t, B * NSB - 1) % NSB,
                                           0)),
                pl.BlockSpec((1, 1, E), lambda t, p: (_bf(t), 0, 0)),
                pl.BlockSpec((B, 1, IDXW), lambda t, p: (0, 0, 0)),
                pl.BlockSpec((E, E), lambda t, p: (0, 0)),
                pl.BlockSpec((E, E), lambda t, p: (1, 0)),
                pl.BlockSpec((E, E), lambda t, p: (2, 0)),
                pl.BlockSpec((E, E), lambda t, p: (0, 0)),
                bspec, bspec, bspec,
            ],
            out_specs=pl.BlockSpec(
                (1, SB, E), lambda t, p: (_bf(t), _sf(t), 0)),
            scratch_shapes=[
                pltpu.VMEM((ROWS, E), BF),
                pltpu.VMEM((SELP, E), jnp.float32),
                pltpu.VMEM((ROWS, E), jnp.float32),
                pltpu.VMEM((ROWS, 1), jnp.float32),
                pltpu.VMEM((ROWS, 1), jnp.float32),
                pltpu.VMEM((E, E), BF),
                pltpu.VMEM((E, E), BF),
                pltpu.VMEM((E, E), BF),
                pltpu.VMEM((E, E), BF),
            ],
        ),
        out_shape=jax.ShapeDtypeStruct((B, S, E), jnp.float32),
    )(idx2d, xq, x, mean, idx, W_qkv, W_qkv, W_qkv, W_out, bq, bv, bo)


def kernel(x, W_qkv, b_qkv, W_out, b_out):
    bq = b_qkv[0:E].reshape(1, E)
    bv = b_qkv[2 * E:3 * E].reshape(1, E)
    bo = b_out.reshape(1, E)

    mean, idx = _stats(x)
    idx2d = idx.reshape(B, IDXW)
    xq = _sc_gather(x, idx2d)
    return _mega(xq, idx2d, mean, idx, W_qkv, W_out, bq, bv, bo, x)


# speedup vs baseline: 1.1091x; 1.1091x over previous
import math
from functools import partial

import jax
import jax.numpy as jnp
from jax.experimental import pallas as pl
from jax.experimental.pallas import tpu as pltpu

B, S, E, H = 4, 4096, 1024, 16
DH = E // H
SEL = max(1, int(5 * math.log(S + 1)))
SEL = min(SEL, S)
SELP = 48
ROWS = H * SELP
SB = 1024
NSB = S // SB
T_STEPS = B * NSB + NSB
SCALE = 1.0 / math.sqrt(DH)
IDXW = 64
BF = jnp.bfloat16


def _head_mask(rows, cols, row_group, col_group, dtype):
    r = jax.lax.broadcasted_iota(jnp.int32, (rows, cols), 0) // row_group
    c = jax.lax.broadcasted_iota(jnp.int32, (rows, cols), 1) // col_group
    return (r == c).astype(dtype)


def _stats_kernel(x_ref, mean_ref, idx_ref, e_ref, cs_ref):
    b = pl.program_id(0)
    s = pl.program_id(1)
    xb = x_ref[0]
    colsum = jnp.sum(xb, axis=0).reshape(1, E)
    y = xb[:, 0:128] * xb[:, 0:128]
    for k in range(1, E // 128):
        xk = xb[:, k * 128:(k + 1) * 128]
        y = y + xk * xk
    energy = jnp.sum(y, axis=1).reshape(1, SBS)

    @pl.when(s == 0)
    def _():
        cs_ref[...] = colsum

    @pl.when(s != 0)
    def _():
        cs_ref[...] += colsum

    e_ref[pl.ds(b, 1), pl.ds(s * SBS, SBS)] = energy

    @pl.when(s == NSBS - 1)
    def _():
        mean_ref[0] = cs_ref[...] * (1.0 / S)

    @pl.when((b == B - 1) & (s == NSBS - 1))
    def _():
        lane_s = jax.lax.broadcasted_iota(jnp.int32, (B, S), 1)
        lane_w = jax.lax.broadcasted_iota(jnp.int32, (B, IDXW), 1)

        def body(j, carry):
            e, acc = carry
            m = jnp.max(e, axis=1, keepdims=True)
            idxv = jnp.min(jnp.where(e == m, lane_s, S), axis=1, keepdims=True)
            acc = jnp.where(lane_w == j, idxv, acc)
            e = jnp.where(lane_s == idxv, -1.0, e)
            return e, acc

        _, acc = jax.lax.fori_loop(
            0, SEL, body, (e_ref[...], jnp.zeros((B, IDXW), jnp.int32)))
        idx_ref[...] = acc.reshape(B, 1, IDXW)


SBS = 2048
NSBS = S // SBS


def _stats(x):
    return pl.pallas_call(
        _stats_kernel,
        grid=(B, NSBS),
        in_specs=[pl.BlockSpec((1, SBS, E), lambda b, s: (b, s, 0))],
        out_specs=[
            pl.BlockSpec((1, 1, E), lambda b, s: (b, 0, 0)),
            pl.BlockSpec((B, 1, IDXW), lambda b, s: (0, 0, 0)),
        ],
        out_shape=[
            jax.ShapeDtypeStruct((B, 1, E), jnp.float32),
            jax.ShapeDtypeStruct((B, 1, IDXW), jnp.int32),
        ],
        scratch_shapes=[
            pltpu.VMEM((B, S), jnp.float32),
            pltpu.VMEM((1, E), jnp.float32),
        ],
    )(x)


def _bc(t):
    return jnp.minimum(t // NSB, B - 1)


def _bf(t):
    return jnp.maximum(t - NSB, 0) // NSB


def _sf(t):
    return jnp.maximum(t - NSB, 0) % NSB


def _mega_kernel(idxp_ref, xany_ref, x_ref, mean_ref, idx_ref,
                 wq_ref, wk_ref, wv_ref, wo_ref, bq_ref, bv_ref, bo_ref,
                 fill_ref,
                 qt_ref, xq_ref, rows_ref, acc_ref, m_ref, l_ref,
                 wq16_ref, wk16_ref, wv16_ref, wo16_ref, sem):
    t = pl.program_id(0)
    bc = _bc(t)
    s = t % NSB
    compute = t < B * NSB

    @pl.when(t == 0)
    def _():
        wq16_ref[...] = wq_ref[...].astype(BF)
        wk16_ref[...] = wk_ref[...].astype(BF)
        wv16_ref[...] = wv_ref[...].astype(BF)
        wo16_ref[...] = wo_ref[...].astype(BF)

    @pl.when(t >= NSB)
    def _():
        bf = _bf(t)
        sf = _sf(t)
        mean = mean_ref[0]
        delta = rows_ref[...] - mean
        row_g = jax.lax.broadcasted_iota(jnp.int32, (SB, SELP), 0) + sf * SB
        col_j = jax.lax.broadcasted_iota(jnp.int32, (SB, SELP), 1)
        idxv = idx_ref[bf, 0, :SELP].reshape(1, SELP)
        onehot = ((row_g == idxv) & (col_j < SEL)).astype(BF)
        fill_ref[0] = jnp.broadcast_to(mean, (SB, E)) + jax.lax.dot_general(
            onehot, delta.astype(BF), (((1,), (0,)), ((), ())),
            preferred_element_type=jnp.float32)

    @pl.when(compute & (s == 0))
    def _():
        copies = []
        for i in range(SELP):
            r = idxp_ref[bc, i]
            c = pltpu.make_async_copy(
                xany_ref.at[bc, pl.ds(r, 1), :], xq_ref.at[pl.ds(i, 1), :],
                sem)
            c.start()
            copies.append(c)
        for c in copies:
            c.wait()
        q = jax.lax.dot_general(
            xq_ref[...].astype(BF), wq16_ref[...], (((1,), (1,)), ((), ())),
            preferred_element_type=jnp.float32) + bq_ref[...]
        qe = jnp.concatenate([q] * H, axis=0)
        qe = (qe * _head_mask(ROWS, E, SELP, DH, jnp.float32)).astype(BF)
        qt = jax.lax.dot_general(
            qe, wk16_ref[...], (((1,), (0,)), ((), ())),
            preferred_element_type=jnp.float32)
        qt_ref[...] = (qt * SCALE).astype(BF)

    @pl.when(compute)
    def _():
        xb16 = x_ref[0].astype(BF)
        scores = jax.lax.dot_general(
            qt_ref[...], xb16, (((1,), (1,)), ((), ())),
            preferred_element_type=jnp.float32)
        bmax = jnp.max(scores, axis=1, keepdims=True)

        @pl.when(s == 0)
        def _():
            p = jnp.exp(scores - bmax)
            m_ref[...] = bmax
            l_ref[...] = jnp.sum(p, axis=1, keepdims=True)
            acc_ref[...] = jax.lax.dot_general(
                p.astype(BF), xb16, (((1,), (0,)), ((), ())),
                preferred_element_type=jnp.float32)

        @pl.when(s != 0)
        def _():
            m_old = m_ref[...]
            m_new = jnp.maximum(m_old, bmax)
            alpha = jnp.exp(m_old - m_new)
            p = jnp.exp(scores - m_new)
            m_ref[...] = m_new
            l_ref[...] = l_ref[...] * alpha + jnp.sum(p, axis=1, keepdims=True)
            acc_ref[...] = acc_ref[...] * alpha + jax.lax.dot_general(
                p.astype(BF), xb16, (((1,), (0,)), ((), ())),
                preferred_element_type=jnp.float32)

        @pl.when(s == NSB - 1)
        def _():
            z = acc_ref[...] / l_ref[...]
            oh = jax.lax.dot_general(
                z.astype(BF), wv16_ref[...], (((1,), (1,)), ((), ())),
                preferred_element_type=jnp.float32)
            oh = oh * _head_mask(ROWS, E, SELP, DH, jnp.float32)
            folded = jnp.zeros((SELP, E), jnp.float32)
            for h in range(H):
                folded = folded + oh[h * SELP:(h + 1) * SELP, :]
            bvo = jax.lax.dot_general(
                bv_ref[...].astype(BF), wo16_ref[...], (((1,), (1,)), ((), ())),
                preferred_element_type=jnp.float32) + bo_ref[...]
            rows_ref[...] = jax.lax.dot_general(
                folded.astype(BF), wo16_ref[...], (((1,), (1,)), ((), ())),
                preferred_element_type=jnp.float32) + bvo


def _mega(x, idx2d, mean, idx, W_qkv, W_out, bq, bv, bo):
    bspec = pl.BlockSpec((1, E), lambda t, p: (0, 0))
    return pl.pallas_call(
        _mega_kernel,
        grid_spec=pltpu.PrefetchScalarGridSpec(
            num_scalar_prefetch=1,
            grid=(T_STEPS,),
            in_specs=[
                pl.BlockSpec(memory_space=pl.ANY),
                pl.BlockSpec((1, SB, E),
                             lambda t, p: (_bc(t),
                                           jnp.minimum(t, B * NSB - 1) % NSB,
                                           0)),
                pl.BlockSpec((1, 1, E), lambda t, p: (_bf(t), 0, 0)),
                pl.BlockSpec((B, 1, IDXW), lambda t, p: (0, 0, 0)),
                pl.BlockSpec((E, E), lambda t, p: (0, 0)),
                pl.BlockSpec((E, E), lambda t, p: (1, 0)),
                pl.BlockSpec((E, E), lambda t, p: (2, 0)),
                pl.BlockSpec((E, E), lambda t, p: (0, 0)),
                bspec, bspec, bspec,
            ],
            out_specs=pl.BlockSpec(
                (1, SB, E), lambda t, p: (_bf(t), _sf(t), 0)),
            scratch_shapes=[
                pltpu.VMEM((ROWS, E), BF),
                pltpu.VMEM((SELP, E), jnp.float32),
                pltpu.VMEM((SELP, E), jnp.float32),
                pltpu.VMEM((ROWS, E), jnp.float32),
                pltpu.VMEM((ROWS, 1), jnp.float32),
                pltpu.VMEM((ROWS, 1), jnp.float32),
                pltpu.VMEM((E, E), BF),
                pltpu.VMEM((E, E), BF),
                pltpu.VMEM((E, E), BF),
                pltpu.VMEM((E, E), BF),
                pltpu.SemaphoreType.DMA,
            ],
        ),
        out_shape=jax.ShapeDtypeStruct((B, S, E), jnp.float32),
    )(idx2d, x, x, mean, idx, W_qkv, W_qkv, W_qkv, W_out, bq, bv, bo)


def kernel(x, W_qkv, b_qkv, W_out, b_out):
    bq = b_qkv[0:E].reshape(1, E)
    bv = b_qkv[2 * E:3 * E].reshape(1, E)
    bo = b_out.reshape(1, E)

    mean, idx = _stats(x)
    idx2d = idx.reshape(B, IDXW)
    return _mega(x, idx2d, mean, idx, W_qkv, W_out, bq, bv, bo)


# speedup vs baseline: 1.1109x; 1.0016x over previous
import math

import jax
import jax.numpy as jnp
from jax.experimental import pallas as pl
from jax.experimental.pallas import tpu as pltpu

B, S, E, H = 4, 4096, 1024, 16
DH = E // H
SEL = max(1, int(5 * math.log(S + 1)))
SEL = min(SEL, S)
SELP = 48
ROWS = H * SELP
SB = 1024
NSB = S // SB
T_STEPS = B * NSB + NSB
SCALE = 1.0 / math.sqrt(DH)
IDXW = 64
BF = jnp.bfloat16


def _head_mask(rows, cols, row_group, col_group, dtype):
    r = jax.lax.broadcasted_iota(jnp.int32, (rows, cols), 0) // row_group
    c = jax.lax.broadcasted_iota(jnp.int32, (rows, cols), 1) // col_group
    return (r == c).astype(dtype)


def _stats_kernel(x_ref, mean_ref, idx_ref, e_ref, cs_ref):
    b = pl.program_id(0)
    s = pl.program_id(1)
    xb = x_ref[0]
    colsum = jnp.sum(xb, axis=0).reshape(1, E)
    y = xb[:, 0:128] * xb[:, 0:128]
    for k in range(1, E // 128):
        xk = xb[:, k * 128:(k + 1) * 128]
        y = y + xk * xk
    energy = jnp.sum(y, axis=1).reshape(1, SBS)

    @pl.when(s == 0)
    def _():
        cs_ref[...] = colsum

    @pl.when(s != 0)
    def _():
        cs_ref[...] += colsum

    e_ref[pl.ds(b, 1), pl.ds(s * SBS, SBS)] = energy

    @pl.when(s == NSBS - 1)
    def _():
        mean_ref[0] = cs_ref[...] * (1.0 / S)

    @pl.when((b == B - 1) & (s == NSBS - 1))
    def _():
        lane_s = jax.lax.broadcasted_iota(jnp.int32, (B, S), 1)
        lane_w = jax.lax.broadcasted_iota(jnp.int32, (B, IDXW), 1)

        def body(j, carry):
            e, acc = carry
            m = jnp.max(e, axis=1, keepdims=True)
            idxv = jnp.min(jnp.where(e == m, lane_s, S), axis=1, keepdims=True)
            acc = jnp.where(lane_w == j, idxv, acc)
            e = jnp.where(lane_s == idxv, -1.0, e)
            return e, acc

        _, acc = jax.lax.fori_loop(
            0, SEL, body, (e_ref[...], jnp.zeros((B, IDXW), jnp.int32)))
        idx_ref[...] = acc.reshape(B, 1, IDXW)


SBS = 2048
NSBS = S // SBS


def _stats(x):
    return pl.pallas_call(
        _stats_kernel,
        grid=(B, NSBS),
        in_specs=[pl.BlockSpec((1, SBS, E), lambda b, s: (b, s, 0))],
        out_specs=[
            pl.BlockSpec((1, 1, E), lambda b, s: (b, 0, 0)),
            pl.BlockSpec((B, 1, IDXW), lambda b, s: (0, 0, 0)),
        ],
        out_shape=[
            jax.ShapeDtypeStruct((B, 1, E), jnp.float32),
            jax.ShapeDtypeStruct((B, 1, IDXW), jnp.int32),
        ],
        scratch_shapes=[
            pltpu.VMEM((B, S), jnp.float32),
            pltpu.VMEM((1, E), jnp.float32),
        ],
    )(x)


def _bc(t):
    return jnp.minimum(t // NSB, B - 1)


def _bf(t):
    return jnp.maximum(t - NSB, 0) // NSB


def _sf(t):
    return jnp.maximum(t - NSB, 0) % NSB


def _mega_kernel(idxp_ref, xany_ref, x_ref, mean_ref, idx_ref,
                 wq_ref, wk_ref, wv_ref, wo_ref, bq_ref, bv_ref, bo_ref,
                 fill_ref,
                 qt_ref, xq_ref, rows_ref, acc_ref, m_ref, l_ref,
                 wq16_ref, wk16_ref, wv16_ref, wo16_ref, sem):
    t = pl.program_id(0)
    bc = _bc(t)
    s = t % NSB
    compute = t < B * NSB

    @pl.when(t == 0)
    def _():
        wq16_ref[...] = wq_ref[...].astype(BF)
        wk16_ref[...] = wk_ref[...].astype(BF)
        wv16_ref[...] = wv_ref[...].astype(BF)
        wo16_ref[...] = wo_ref[...].astype(BF)

    @pl.when(t >= NSB)
    def _():
        bf = _bf(t)
        sf = _sf(t)
        mean = mean_ref[0]
        delta = rows_ref[...] - mean
        row_g = jax.lax.broadcasted_iota(jnp.int32, (SB, SELP), 0) + sf * SB
        col_j = jax.lax.broadcasted_iota(jnp.int32, (SB, SELP), 1)
        idxv = idx_ref[bf, 0, :SELP].reshape(1, SELP)
        onehot = ((row_g == idxv) & (col_j < SEL)).astype(BF)
        fill_ref[0] = jnp.broadcast_to(mean, (SB, E)) + jax.lax.dot_general(
            onehot, delta.astype(BF), (((1,), (0,)), ((), ())),
            preferred_element_type=jnp.float32)

    @pl.when(compute & (s == 0))
    def _():
        copies = []
        for i in range(SELP):
            r = idxp_ref[bc, i]
            c = pltpu.make_async_copy(
                xany_ref.at[bc, pl.ds(r, 1), :], xq_ref.at[pl.ds(i, 1), :],
                sem)
            c.start()
            copies.append(c)
        for c in copies:
            c.wait()
        q = jax.lax.dot_general(
            xq_ref[...].astype(BF), wq16_ref[...], (((1,), (1,)), ((), ())),
            preferred_element_type=jnp.float32) + bq_ref[...]
        qe = jnp.concatenate([q] * H, axis=0)
        qe = (qe * _head_mask(ROWS, E, SELP, DH, jnp.float32)).astype(BF)
        qt = jax.lax.dot_general(
            qe, wk16_ref[...], (((1,), (0,)), ((), ())),
            preferred_element_type=jnp.float32)
        qt_ref[...] = (qt * SCALE).astype(BF)

    @pl.when(compute)
    def _():
        xb16 = x_ref[0].astype(BF)
        scores = jax.lax.dot_general(
            qt_ref[...], xb16, (((1,), (1,)), ((), ())),
            preferred_element_type=jnp.float32)
        bmax = jnp.max(scores, axis=1, keepdims=True)

        @pl.when(s == 0)
        def _():
            p = jnp.exp(scores - bmax)
            m_ref[...] = bmax
            l_ref[...] = jnp.sum(p, axis=1, keepdims=True)
            acc_ref[...] = jax.lax.dot_general(
                p.astype(BF), xb16, (((1,), (0,)), ((), ())),
                preferred_element_type=jnp.float32)

        @pl.when(s != 0)
        def _():
            m_old = m_ref[...]
            m_new = jnp.maximum(m_old, bmax)
            alpha = jnp.exp(m_old - m_new)
            p = jnp.exp(scores - m_new)
            m_ref[...] = m_new
            l_ref[...] = l_ref[...] * alpha + jnp.sum(p, axis=1, keepdims=True)
            acc_ref[...] = acc_ref[...] * alpha + jax.lax.dot_general(
                p.astype(BF), xb16, (((1,), (0,)), ((), ())),
                preferred_element_type=jnp.float32)

        @pl.when(s == NSB - 1)
        def _():
            z = acc_ref[...] / l_ref[...]
            oh = jax.lax.dot_general(
                z.astype(BF), wv16_ref[...], (((1,), (1,)), ((), ())),
                preferred_element_type=jnp.float32)
            oh = oh * _head_mask(ROWS, E, SELP, DH, jnp.float32)
            folded = jnp.zeros((SELP, E), jnp.float32)
            for h in range(H):
                folded = folded + oh[h * SELP:(h + 1) * SELP, :]
            bvo = jax.lax.dot_general(
                bv_ref[...].astype(BF), wo16_ref[...], (((1,), (1,)), ((), ())),
                preferred_element_type=jnp.float32) + bo_ref[...]
            rows_ref[...] = jax.lax.dot_general(
                folded.astype(BF), wo16_ref[...], (((1,), (1,)), ((), ())),
                preferred_element_type=jnp.float32) + bvo


def _mega(x, idx2d, mean, idx, W_qkv, W_out, bq, bv, bo):
    bspec = pl.BlockSpec((1, E), lambda t, p: (0, 0))
    return pl.pallas_call(
        _mega_kernel,
        grid_spec=pltpu.PrefetchScalarGridSpec(
            num_scalar_prefetch=1,
            grid=(T_STEPS,),
            in_specs=[
                pl.BlockSpec(memory_space=pl.ANY),
                pl.BlockSpec((1, SB, E),
                             lambda t, p: (_bc(t),
                                           jnp.minimum(t, B * NSB - 1) % NSB,
                                           0)),
                pl.BlockSpec((1, 1, E), lambda t, p: (_bf(t), 0, 0)),
                pl.BlockSpec((B, 1, IDXW), lambda t, p: (0, 0, 0)),
                pl.BlockSpec((E, E), lambda t, p: (0, 0)),
                pl.BlockSpec((E, E), lambda t, p: (1, 0)),
                pl.BlockSpec((E, E), lambda t, p: (2, 0)),
                pl.BlockSpec((E, E), lambda t, p: (0, 0)),
                bspec, bspec, bspec,
            ],
            out_specs=pl.BlockSpec(
                (1, SB, E), lambda t, p: (_bf(t), _sf(t), 0)),
            scratch_shapes=[
                pltpu.VMEM((ROWS, E), BF),
                pltpu.VMEM((SELP, E), jnp.float32),
                pltpu.VMEM((SELP, E), jnp.float32),
                pltpu.VMEM((ROWS, E), jnp.float32),
                pltpu.VMEM((ROWS, 1), jnp.float32),
                pltpu.VMEM((ROWS, 1), jnp.float32),
                pltpu.VMEM((E, E), BF),
                pltpu.VMEM((E, E), BF),
                pltpu.VMEM((E, E), BF),
                pltpu.VMEM((E, E), BF),
                pltpu.SemaphoreType.DMA,
            ],
        ),
        out_shape=jax.ShapeDtypeStruct((B, S, E), jnp.float32),
    )(idx2d, x, x, mean, idx, W_qkv, W_qkv, W_qkv, W_out, bq, bv, bo)


def kernel(x, W_qkv, b_qkv, W_out, b_out):
    bq = b_qkv[0:E].reshape(1, E)
    bv = b_qkv[2 * E:3 * E].reshape(1, E)
    bo = b_out.reshape(1, E)

    mean, idx = _stats(x)
    idx2d = idx.reshape(B, IDXW)
    return _mega(x, idx2d, mean, idx, W_qkv, W_out, bq, bv, bo)
